# Initial kernel scaffold; baseline (speedup 1.0000x reference)
import jax
import jax.numpy as jnp
from jax.experimental import pallas as pl


def kernel(x, edge_index, Wrel0, brel0, Wroot0, Wrel1, brel1, Wroot1, Wrel2, brel2, Wroot2, Wg, bg):
    raise NotImplementedError("write your pallas kernel here")


# speedup vs baseline: 2.6794x; 2.6794x over previous
import functools

import jax
import jax.numpy as jnp
from jax import lax
from jax.experimental import pallas as pl
from jax.experimental.pallas import tpu as pltpu
from jax.experimental.pallas import tpu_sc as plsc

N_NODES = 10000
D = 128
NC = 2
NS = 16
K = 80
HALF = 5120
TBL = 5248
TRASH = HALF
TBL_PER_TILE = TBL // NS
OUT_PER_TILE = HALF // NS
N_PAD = 2 * HALF


def _agg_body(h_hbm, src_hbm, dst_hbm, out_hbm, src_v, dst_v, rows_v, zbuf_v,
              sem, agg_sh, *, n_edges):
  c = lax.axis_index("c")
  s = lax.axis_index("s")
  edges_per_tile = n_edges // NS
  n_chunks = edges_per_tile // K
  half_base = c * HALF

  zeros16 = jnp.zeros((16,), jnp.float32)

  def zrow(i, _):
    for j in range(8):
      zbuf_v[i, pl.ds(j * 16, 16)] = zeros16
    return 0

  lax.fori_loop(0, TBL_PER_TILE, zrow, 0)
  pltpu.sync_copy(zbuf_v.at[pl.ds(0, TBL_PER_TILE)],
                  agg_sh.at[pl.ds(s * TBL_PER_TILE, TBL_PER_TILE)])
  plsc.subcore_barrier()

  trash16 = jnp.full((16,), TRASH, jnp.int32)

  def chunk(i, _):
    base = s * edges_per_tile + i * K
    pltpu.sync_copy(src_hbm.at[pl.ds(base, K)], src_v)
    pltpu.sync_copy(dst_hbm.at[pl.ds(base, K)], dst_v)
    for g in range(K // 16):
      d = dst_v[pl.ds(g * 16, 16)]
      loc = d - half_base
      ok = (loc >= 0) & (loc < HALF)
      dst_v[pl.ds(g * 16, 16)] = jnp.where(ok, loc, trash16)
    pltpu.async_copy(h_hbm.at[src_v], rows_v, sem).wait()
    pltpu.sync_copy(rows_v, agg_sh.at[dst_v], add=True)
    return 0

  lax.fori_loop(0, n_chunks, chunk, 0)
  plsc.subcore_barrier()

  pltpu.sync_copy(agg_sh.at[pl.ds(s * OUT_PER_TILE, OUT_PER_TILE)],
                  zbuf_v.at[pl.ds(0, OUT_PER_TILE)])
  pltpu.sync_copy(zbuf_v.at[pl.ds(0, OUT_PER_TILE)],
                  out_hbm.at[pl.ds(half_base + s * OUT_PER_TILE, OUT_PER_TILE)])


def _make_agg(n_edges):
  mesh = plsc.VectorSubcoreMesh(core_axis_name="c", subcore_axis_name="s",
                                num_cores=NC)
  return pl.kernel(
      functools.partial(_agg_body, n_edges=n_edges),
      out_type=jax.ShapeDtypeStruct((N_PAD, D), jnp.float32),
      mesh=mesh,
      scratch_types=[
          pltpu.VMEM((K,), jnp.int32),
          pltpu.VMEM((K,), jnp.int32),
          pltpu.VMEM((K, D), jnp.float32),
          pltpu.VMEM((TBL_PER_TILE, D), jnp.float32),
          pltpu.SemaphoreType.DMA,
          pltpu.VMEM_SHARED((TBL, D), jnp.float32),
      ],
  )


def _dense_mid_body(a_ref, h_ref, wrelT_ref, wrootT_ref, b_ref, o_ref):
  y = jnp.dot(a_ref[...], wrelT_ref[...], preferred_element_type=jnp.float32)
  y += jnp.dot(h_ref[...], wrootT_ref[...], preferred_element_type=jnp.float32)
  y += b_ref[...]
  o_ref[...] = jnp.maximum(y, 0.0)


def _final_body(h_ref, wgT_ref, bg_ref, o_ref):
  o_ref[...] = (
      jnp.dot(h_ref[...], wgT_ref[...], preferred_element_type=jnp.float32)
      + bg_ref[...])


_R = 2000


def _dense_mid(agg, h, wrelT, wrootT, b2d):
  grid = (N_NODES // _R,)
  return pl.pallas_call(
      _dense_mid_body,
      grid=grid,
      in_specs=[
          pl.BlockSpec((_R, D), lambda i: (i, 0)),
          pl.BlockSpec((_R, D), lambda i: (i, 0)),
          pl.BlockSpec((D, D), lambda i: (0, 0)),
          pl.BlockSpec((D, D), lambda i: (0, 0)),
          pl.BlockSpec((1, D), lambda i: (0, 0)),
      ],
      out_specs=pl.BlockSpec((_R, D), lambda i: (i, 0)),
      out_shape=jax.ShapeDtypeStruct((N_NODES, D), jnp.float32),
  )(agg, h, wrelT, wrootT, b2d)


def _final(h, wgT, bg2d):
  grid = (N_NODES // _R,)
  return pl.pallas_call(
      _final_body,
      grid=grid,
      in_specs=[
          pl.BlockSpec((_R, D), lambda i: (i, 0)),
          pl.BlockSpec((D, D), lambda i: (0, 0)),
          pl.BlockSpec((1, D), lambda i: (0, 0)),
      ],
      out_specs=pl.BlockSpec((_R, D), lambda i: (i, 0)),
      out_shape=jax.ShapeDtypeStruct((N_NODES, D), jnp.float32),
  )(h, wgT, bg2d)


def kernel(x, edge_index, Wrel0, brel0, Wroot0, Wrel1, brel1, Wroot1, Wrel2,
           brel2, Wroot2, Wg, bg):
  n_edges = edge_index.shape[1]
  src = edge_index[0]
  dst = edge_index[1]
  agg_fn = _make_agg(n_edges)

  wrelT = jnp.stack([Wrel0.T, Wrel1.T, Wrel2.T])
  wrootT = jnp.stack([Wroot0.T, Wroot1.T, Wroot2.T])
  b2 = jnp.stack([brel0.reshape(1, D), brel1.reshape(1, D),
                  brel2.reshape(1, D)])

  def layer(h, ws):
    wrelT_i, wrootT_i, b_i = ws
    agg = agg_fn(h, src, dst)
    h2 = _dense_mid(agg, h, wrelT_i, wrootT_i, b_i)
    return h2, None

  h3, _ = lax.scan(layer, x, (wrelT, wrootT, b2))
  return _final(h3, Wg.T, bg.reshape(1, D))


# speedup vs baseline: 5.2282x; 1.9513x over previous
import functools

import jax
import jax.numpy as jnp
from jax import lax
from jax.experimental import pallas as pl
from jax.experimental.pallas import tpu as pltpu
from jax.experimental.pallas import tpu_sc as plsc

N_NODES = 10000
D = 128
NC = 2
NS = 16
K = 80
HALF = 5120
TBL = 5248
TRASH = HALF
TBL_PER_TILE = TBL // NS
OUT_PER_TILE = HALF // NS
N_PAD = 2 * HALF


def _agg_body(h_hbm, src_hbm, dst_hbm, out_hbm, src_blk, dst_blk, dstc0, dstc1,
              rows0, rows1, sem0, sem1, agg_sh, *, n_edges):
  c = lax.axis_index("c")
  s = lax.axis_index("s")
  edges_per_tile = n_edges // NS
  n_chunks = edges_per_tile // K
  half_base = c * HALF
  dstc = (dstc0, dstc1)
  rows = (rows0, rows1)
  sems = (sem0, sem1)

  zeros16 = jnp.zeros((16,), jnp.float32)

  def zrow(i, _):
    for j in range(8):
      rows0[i, pl.ds(j * 16, 16)] = zeros16
    return 0

  pltpu.async_copy(src_hbm.at[pl.ds(s * edges_per_tile, edges_per_tile)],
                   src_blk, sem0)
  pltpu.async_copy(dst_hbm.at[pl.ds(s * edges_per_tile, edges_per_tile)],
                   dst_blk, sem1)
  lax.fori_loop(0, K, zrow, 0)
  for q in range(4):
    pltpu.sync_copy(rows0,
                    agg_sh.at[pl.ds(s * TBL_PER_TILE + q * K, K)])
  pltpu.sync_copy(rows0.at[pl.ds(0, TBL_PER_TILE - 4 * K)],
                  agg_sh.at[pl.ds(s * TBL_PER_TILE + 4 * K,
                                  TBL_PER_TILE - 4 * K)])
  pltpu.make_async_copy(
      src_hbm.at[pl.ds(s * edges_per_tile, edges_per_tile)], src_blk,
      sem0).wait()
  pltpu.make_async_copy(
      dst_hbm.at[pl.ds(s * edges_per_tile, edges_per_tile)], dst_blk,
      sem1).wait()
  plsc.subcore_barrier()

  trash16 = jnp.full((16,), TRASH, jnp.int32)

  def localize(g, b):
    for j in range(K // 16):
      d = dst_blk[pl.ds(g * K + j * 16, 16)]
      loc = d - half_base
      ok = (loc >= 0) & (loc < HALF)
      dstc[b][pl.ds(j * 16, 16)] = jnp.where(ok, loc, trash16)

  def start_gather(g, b):
    pltpu.async_copy(h_hbm.at[src_blk.at[pl.ds(g * K, K)]], rows[b], sems[b])

  def wait_gather(g, b):
    pltpu.make_async_copy(h_hbm.at[src_blk.at[pl.ds(g * K, K)]], rows[b],
                          sems[b]).wait()

  localize(0, 0)
  start_gather(0, 0)

  def pair(p, _):
    for b in range(2):
      g = 2 * p + b
      wait_gather(g, b)

      @pl.when(g + 1 < n_chunks)
      def _():
        localize(g + 1, 1 - b)
        start_gather(g + 1, 1 - b)

      pltpu.sync_copy(rows[b], agg_sh.at[dstc[b]], add=True)
    return 0

  lax.fori_loop(0, n_chunks // 2, pair, 0)
  plsc.subcore_barrier()

  for q in range(4):
    buf = rows0 if q % 2 == 0 else rows1
    pltpu.sync_copy(agg_sh.at[pl.ds(s * OUT_PER_TILE + q * K, K)], buf)
    pltpu.sync_copy(
        buf, out_hbm.at[pl.ds(half_base + s * OUT_PER_TILE + q * K, K)])


def _make_agg(n_edges):
  mesh = plsc.VectorSubcoreMesh(core_axis_name="c", subcore_axis_name="s",
                                num_cores=NC)
  ept = n_edges // NS
  return pl.kernel(
      functools.partial(_agg_body, n_edges=n_edges),
      out_type=jax.ShapeDtypeStruct((N_PAD, D), jnp.float32),
      mesh=mesh,
      scratch_types=[
          pltpu.VMEM((ept,), jnp.int32),
          pltpu.VMEM((ept,), jnp.int32),
          pltpu.VMEM((K,), jnp.int32),
          pltpu.VMEM((K,), jnp.int32),
          pltpu.VMEM((K, D), jnp.float32),
          pltpu.VMEM((K, D), jnp.float32),
          pltpu.SemaphoreType.DMA,
          pltpu.SemaphoreType.DMA,
          pltpu.VMEM_SHARED((TBL, D), jnp.float32),
      ],
  )


def _dense_mid_body(a_ref, h_ref, wrelT_ref, wrootT_ref, b_ref, o_ref):
  y = jnp.dot(a_ref[...], wrelT_ref[...], preferred_element_type=jnp.float32)
  y += jnp.dot(h_ref[...], wrootT_ref[...], preferred_element_type=jnp.float32)
  y += b_ref[...]
  o_ref[...] = jnp.maximum(y, 0.0)


def _final_body(h_ref, wgT_ref, bg_ref, o_ref):
  o_ref[...] = (
      jnp.dot(h_ref[...], wgT_ref[...], preferred_element_type=jnp.float32)
      + bg_ref[...])


_R = 2000


def _dense_mid(agg, h, wrelT, wrootT, b2d):
  grid = (N_NODES // _R,)
  return pl.pallas_call(
      _dense_mid_body,
      grid=grid,
      in_specs=[
          pl.BlockSpec((_R, D), lambda i: (i, 0)),
          pl.BlockSpec((_R, D), lambda i: (i, 0)),
          pl.BlockSpec((D, D), lambda i: (0, 0)),
          pl.BlockSpec((D, D), lambda i: (0, 0)),
          pl.BlockSpec((1, D), lambda i: (0, 0)),
      ],
      out_specs=pl.BlockSpec((_R, D), lambda i: (i, 0)),
      out_shape=jax.ShapeDtypeStruct((N_NODES, D), jnp.float32),
  )(agg, h, wrelT, wrootT, b2d)


def _final(h, wgT, bg2d):
  grid = (N_NODES // _R,)
  return pl.pallas_call(
      _final_body,
      grid=grid,
      in_specs=[
          pl.BlockSpec((_R, D), lambda i: (i, 0)),
          pl.BlockSpec((D, D), lambda i: (0, 0)),
          pl.BlockSpec((1, D), lambda i: (0, 0)),
      ],
      out_specs=pl.BlockSpec((_R, D), lambda i: (i, 0)),
      out_shape=jax.ShapeDtypeStruct((N_NODES, D), jnp.float32),
  )(h, wgT, bg2d)


def kernel(x, edge_index, Wrel0, brel0, Wroot0, Wrel1, brel1, Wroot1, Wrel2,
           brel2, Wroot2, Wg, bg):
  n_edges = edge_index.shape[1]
  src = edge_index[0]
  dst = edge_index[1]
  agg_fn = _make_agg(n_edges)

  wrelT = jnp.stack([Wrel0.T, Wrel1.T, Wrel2.T])
  wrootT = jnp.stack([Wroot0.T, Wroot1.T, Wroot2.T])
  b2 = jnp.stack([brel0.reshape(1, D), brel1.reshape(1, D),
                  brel2.reshape(1, D)])

  def layer(h, ws):
    wrelT_i, wrootT_i, b_i = ws
    agg = agg_fn(h, src, dst)
    h2 = _dense_mid(agg, h, wrelT_i, wrootT_i, b_i)
    return h2, None

  h3, _ = lax.scan(layer, x, (wrelT, wrootT, b2))
  return _final(h3, Wg.T, bg.reshape(1, D))


# speedup vs baseline: 7.9812x; 1.5266x over previous
import functools

import jax
import jax.numpy as jnp
from jax import lax
from jax.experimental import pallas as pl
from jax.experimental.pallas import tpu as pltpu
from jax.experimental.pallas import tpu_sc as plsc

N_NODES = 10000
D = 128
NC = 2
NS = 16
K = 80
HALF = 5120
TBL = 5248
TRASH = HALF
TBL_PER_TILE = TBL // NS
OUT_PER_TILE = HALF // NS
N_PAD = 2 * HALF

NV = 64
EPV = 320000 // NV
CAP = 5200
DUMP = 5088
CNTPOS = 5184
BPT = NV // NS
NGRP = 5120 // 16


def _prefix16(x):
  lanes = lax.iota(jnp.int32, 16)
  p = x
  for sh in (1, 2, 4, 8):
    idx = jnp.maximum(lanes - sh, 0)
    g = p.at[idx].get(mode="promise_in_bounds")
    p = p + jnp.where(lanes >= sh, g, 0)
  return p


def _part_body(src_hbm, dst_hbm, srcp_hbm, dstp_hbm, src_blk, dst_blk,
               pos_st, vsrc_st, vdst_st, bsrc_sh, bdst_sh):
  c = lax.axis_index("c")
  s = lax.axis_index("s")
  half_base = c * HALF
  lanes = lax.iota(jnp.int32, 16)
  neg16 = jnp.full((16,), -1, jnp.int32)
  trash16 = jnp.full((16,), TRASH, jnp.int32)

  for j in range(BPT):
    v = BPT * s + j
    abs_base = v * CAP
    pltpu.sync_copy(src_hbm.at[pl.ds(v * EPV, EPV)],
                    src_blk.at[pl.ds(0, EPV)])
    pltpu.sync_copy(dst_hbm.at[pl.ds(v * EPV, EPV)],
                    dst_blk.at[pl.ds(0, EPV)])
    d = dst_blk[pl.ds(EPV - 8, 16)]
    dst_blk[pl.ds(EPV - 8, 16)] = jnp.where(lanes < 8, d, neg16)
    for q in range(EPV + 8, NGRP * 16, 16):
      dst_blk[pl.ds(q, 16)] = neg16

    def batch(bi, cnt):
      for gi in range(8):
        g = bi * 8 + gi
        s16 = src_blk[pl.ds(g * 16, 16)]
        d16 = dst_blk[pl.ds(g * 16, 16)]
        loc = d16 - half_base
        ok = (loc >= 0) & (loc < HALF)
        ok_i = jnp.where(ok, jnp.int32(1), jnp.int32(0))
        pref = _prefix16(ok_i)
        pos = jnp.where(ok, abs_base + cnt + pref - 1,
                        abs_base + DUMP + lanes)
        pos_st[pl.ds(gi * 16, 16)] = pos
        vsrc_st[pl.ds(gi * 16, 16)] = s16
        vdst_st[pl.ds(gi * 16, 16)] = loc
        cnt = cnt + pref[15]
      pltpu.sync_copy(vsrc_st, bsrc_sh.at[pos_st])
      pltpu.sync_copy(vdst_st, bdst_sh.at[pos_st])
      return cnt

    cnt = lax.fori_loop(0, NGRP // 8, batch, jnp.int32(0))

    nch = (cnt + (K - 1)) // K
    for gi in range(8):
      if gi < 5:
        pos_st[pl.ds(gi * 16, 16)] = abs_base + cnt + gi * 16 + lanes
        vsrc_st[pl.ds(gi * 16, 16)] = lanes
        vdst_st[pl.ds(gi * 16, 16)] = trash16
      elif gi == 5:
        pos_st[pl.ds(gi * 16, 16)] = abs_base + CNTPOS + lanes
        vsrc_st[pl.ds(gi * 16, 16)] = lanes
        vdst_st[pl.ds(gi * 16, 16)] = jnp.zeros((16,), jnp.int32) + nch
      else:
        pos_st[pl.ds(gi * 16, 16)] = abs_base + DUMP + lanes
        vsrc_st[pl.ds(gi * 16, 16)] = lanes
        vdst_st[pl.ds(gi * 16, 16)] = trash16
    pltpu.sync_copy(vsrc_st, bsrc_sh.at[pos_st])
    pltpu.sync_copy(vdst_st, bdst_sh.at[pos_st])

  for j in range(BPT):
    v = BPT * s + j
    pltpu.sync_copy(bsrc_sh.at[pl.ds(v * CAP, CAP)], src_blk)
    pltpu.sync_copy(src_blk, srcp_hbm.at[c, v])
    pltpu.sync_copy(bdst_sh.at[pl.ds(v * CAP, CAP)], dst_blk)
    pltpu.sync_copy(dst_blk, dstp_hbm.at[c, v])


def _make_part():
  mesh = plsc.VectorSubcoreMesh(core_axis_name="c", subcore_axis_name="s",
                                num_cores=NC)
  return pl.kernel(
      _part_body,
      out_type=(jax.ShapeDtypeStruct((NC, NV, CAP), jnp.int32),
                jax.ShapeDtypeStruct((NC, NV, CAP), jnp.int32)),
      mesh=mesh,
      scratch_types=[
          pltpu.VMEM((CAP,), jnp.int32),
          pltpu.VMEM((CAP,), jnp.int32),
          pltpu.VMEM((128,), jnp.int32),
          pltpu.VMEM((128,), jnp.int32),
          pltpu.VMEM((128,), jnp.int32),
          pltpu.VMEM_SHARED((NV * CAP,), jnp.int32),
          pltpu.VMEM_SHARED((NV * CAP,), jnp.int32),
      ],
  )


def _agg_body(h_hbm, srcp_hbm, dstp_hbm, out_hbm, src_blk, dst_blk,
              dstc0, dstc1, rows0, rows1, sem0, sem1, agg_sh):
  c = lax.axis_index("c")
  s = lax.axis_index("s")
  half_base = c * HALF
  dstc = (dstc0, dstc1)
  rows = (rows0, rows1)
  sems = (sem0, sem1)

  zeros16 = jnp.zeros((16,), jnp.float32)

  def zrow(i, _):
    for j in range(8):
      rows0[i, pl.ds(j * 16, 16)] = zeros16
    return 0

  lax.fori_loop(0, K, zrow, 0)
  for q in range(4):
    pltpu.sync_copy(rows0, agg_sh.at[pl.ds(s * TBL_PER_TILE + q * K, K)])
  pltpu.sync_copy(rows0.at[pl.ds(0, TBL_PER_TILE - 4 * K)],
                  agg_sh.at[pl.ds(s * TBL_PER_TILE + 4 * K,
                                  TBL_PER_TILE - 4 * K)])
  plsc.subcore_barrier()

  def stage(g, b):
    for j in range(K // 16):
      dstc[b][pl.ds(j * 16, 16)] = dst_blk[pl.ds(g * K + j * 16, 16)]

  def start_gather(g, b):
    pltpu.async_copy(h_hbm.at[src_blk.at[pl.ds(g * K, K)]], rows[b], sems[b])

  def wait_gather(g, b):
    pltpu.make_async_copy(h_hbm.at[src_blk.at[pl.ds(g * K, K)]], rows[b],
                          sems[b]).wait()

  for j in range(BPT):
    v = BPT * s + j
    pltpu.sync_copy(srcp_hbm.at[c, v], src_blk)
    pltpu.sync_copy(dstp_hbm.at[c, v], dst_blk)
    n_ch = dst_blk[pl.ds(CNTPOS, 16)][0]

    @pl.when(n_ch > 0)
    def _():
      stage(0, 0)
      start_gather(0, 0)

    def pair(p, _):
      for b in range(2):
        g = 2 * p + b

        @pl.when(g < n_ch)
        def _():
          wait_gather(g, b)

          @pl.when(g + 1 < n_ch)
          def _():
            stage(g + 1, 1 - b)
            start_gather(g + 1, 1 - b)

          pltpu.sync_copy(rows[b], agg_sh.at[dstc[b]], add=True)
      return 0

    lax.fori_loop(0, (n_ch + 1) // 2, pair, 0)

  plsc.subcore_barrier()

  for q in range(4):
    buf = rows0 if q % 2 == 0 else rows1
    pltpu.sync_copy(agg_sh.at[pl.ds(s * OUT_PER_TILE + q * K, K)], buf)
    pltpu.sync_copy(
        buf, out_hbm.at[pl.ds(half_base + s * OUT_PER_TILE + q * K, K)])


def _make_agg():
  mesh = plsc.VectorSubcoreMesh(core_axis_name="c", subcore_axis_name="s",
                                num_cores=NC)
  return pl.kernel(
      _agg_body,
      out_type=jax.ShapeDtypeStruct((N_PAD, D), jnp.float32),
      mesh=mesh,
      scratch_types=[
          pltpu.VMEM((CAP,), jnp.int32),
          pltpu.VMEM((CAP,), jnp.int32),
          pltpu.VMEM((K,), jnp.int32),
          pltpu.VMEM((K,), jnp.int32),
          pltpu.VMEM((K, D), jnp.float32),
          pltpu.VMEM((K, D), jnp.float32),
          pltpu.SemaphoreType.DMA,
          pltpu.SemaphoreType.DMA,
          pltpu.VMEM_SHARED((TBL, D), jnp.float32),
      ],
  )


def _dense_mid_body(a_ref, h_ref, wrelT_ref, wrootT_ref, b_ref, o_ref):
  y = jnp.dot(a_ref[...], wrelT_ref[...], preferred_element_type=jnp.float32)
  y += jnp.dot(h_ref[...], wrootT_ref[...], preferred_element_type=jnp.float32)
  y += b_ref[...]
  o_ref[...] = jnp.maximum(y, 0.0)


def _final_body(h_ref, wgT_ref, bg_ref, o_ref):
  o_ref[...] = (
      jnp.dot(h_ref[...], wgT_ref[...], preferred_element_type=jnp.float32)
      + bg_ref[...])


_R = 2000


def _dense_mid(agg, h, wrelT, wrootT, b2d):
  grid = (N_NODES // _R,)
  return pl.pallas_call(
      _dense_mid_body,
      grid=grid,
      in_specs=[
          pl.BlockSpec((_R, D), lambda i: (i, 0)),
          pl.BlockSpec((_R, D), lambda i: (i, 0)),
          pl.BlockSpec((D, D), lambda i: (0, 0)),
          pl.BlockSpec((D, D), lambda i: (0, 0)),
          pl.BlockSpec((1, D), lambda i: (0, 0)),
      ],
      out_specs=pl.BlockSpec((_R, D), lambda i: (i, 0)),
      out_shape=jax.ShapeDtypeStruct((N_NODES, D), jnp.float32),
  )(agg, h, wrelT, wrootT, b2d)


def _final(h, wgT, bg2d):
  grid = (N_NODES // _R,)
  return pl.pallas_call(
      _final_body,
      grid=grid,
      in_specs=[
          pl.BlockSpec((_R, D), lambda i: (i, 0)),
          pl.BlockSpec((D, D), lambda i: (0, 0)),
          pl.BlockSpec((1, D), lambda i: (0, 0)),
      ],
      out_specs=pl.BlockSpec((_R, D), lambda i: (i, 0)),
      out_shape=jax.ShapeDtypeStruct((N_NODES, D), jnp.float32),
  )(h, wgT, bg2d)


def kernel(x, edge_index, Wrel0, brel0, Wroot0, Wrel1, brel1, Wroot1, Wrel2,
           brel2, Wroot2, Wg, bg):
  n_edges = edge_index.shape[1]
  src = edge_index[0]
  dst = edge_index[1]
  srcp, dstp = _make_part()(src, dst)
  agg_fn = _make_agg()

  wrelT = jnp.stack([Wrel0.T, Wrel1.T, Wrel2.T])
  wrootT = jnp.stack([Wroot0.T, Wroot1.T, Wroot2.T])
  b2 = jnp.stack([brel0.reshape(1, D), brel1.reshape(1, D),
                  brel2.reshape(1, D)])

  def layer(h, ws):
    wrelT_i, wrootT_i, b_i = ws
    agg = agg_fn(h, srcp, dstp)
    h2 = _dense_mid(agg, h, wrelT_i, wrootT_i, b_i)
    return h2, None

  h3, _ = lax.scan(layer, x, (wrelT, wrootT, b2))
  return _final(h3, Wg.T, bg.reshape(1, D))


# speedup vs baseline: 11.3687x; 1.4244x over previous
import functools

import jax
import jax.numpy as jnp
from jax import lax
from jax.experimental import pallas as pl
from jax.experimental.pallas import tpu as pltpu
from jax.experimental.pallas import tpu_sc as plsc

N_NODES = 10000
D = 128
NC = 2
NS = 16
K = 80
HALF = 5120
TBL = 5248
TRASH = HALF
TBL_PER_TILE = TBL // NS
OUT_PER_TILE = HALF // NS
N_PAD = 2 * HALF

NV = 64
EPV = 320000 // NV
CAP = 5200
DUMP = 5088
CNTPOS = 5184
BPT = NV // NS
NGRP = 5120 // 16


def _prefix16(x):
  lanes = lax.iota(jnp.int32, 16)
  p = x
  for sh in (1, 2, 4, 8):
    idx = jnp.maximum(lanes - sh, 0)
    g = p.at[idx].get(mode="promise_in_bounds")
    p = p + jnp.where(lanes >= sh, g, 0)
  return p


def _part_body(src_hbm, dst_hbm, pkp_hbm, src_blk, dst_blk,
               pos_st, vpk_st, bpk_sh):
  c = lax.axis_index("c")
  s = lax.axis_index("s")
  half_base = c * HALF
  lanes = lax.iota(jnp.int32, 16)
  neg16 = jnp.full((16,), -1, jnp.int32)
  trash_pk = lanes + jnp.int32(TRASH << 16)

  for j in range(BPT):
    v = BPT * s + j
    abs_base = v * CAP
    pltpu.sync_copy(src_hbm.at[pl.ds(v * EPV, EPV)],
                    src_blk.at[pl.ds(0, EPV)])
    pltpu.sync_copy(dst_hbm.at[pl.ds(v * EPV, EPV)],
                    dst_blk.at[pl.ds(0, EPV)])
    d = dst_blk[pl.ds(EPV - 8, 16)]
    dst_blk[pl.ds(EPV - 8, 16)] = jnp.where(lanes < 8, d, neg16)
    for q in range(EPV + 8, NGRP * 16, 16):
      dst_blk[pl.ds(q, 16)] = neg16

    def batch(bi, cnt):
      for gi in range(8):
        g = bi * 8 + gi
        s16 = src_blk[pl.ds(g * 16, 16)]
        d16 = dst_blk[pl.ds(g * 16, 16)]
        loc = d16 - half_base
        ok = (loc >= 0) & (loc < HALF)
        ok_i = jnp.where(ok, jnp.int32(1), jnp.int32(0))
        pref = _prefix16(ok_i)
        pos = jnp.where(ok, abs_base + cnt + pref - 1,
                        abs_base + DUMP + lanes)
        pos_st[pl.ds(gi * 16, 16)] = pos
        vpk_st[pl.ds(gi * 16, 16)] = s16 | (loc << 16)
        cnt = cnt + pref[15]
      pltpu.sync_copy(vpk_st, bpk_sh.at[pos_st])
      return cnt

    cnt = lax.fori_loop(0, NGRP // 8, batch, jnp.int32(0))

    nch = (cnt + (K - 1)) // K
    for gi in range(8):
      if gi < 5:
        pos_st[pl.ds(gi * 16, 16)] = abs_base + cnt + gi * 16 + lanes
        vpk_st[pl.ds(gi * 16, 16)] = trash_pk
      elif gi == 5:
        pos_st[pl.ds(gi * 16, 16)] = abs_base + CNTPOS + lanes
        vpk_st[pl.ds(gi * 16, 16)] = jnp.zeros((16,), jnp.int32) + nch
      else:
        pos_st[pl.ds(gi * 16, 16)] = abs_base + DUMP + lanes
        vpk_st[pl.ds(gi * 16, 16)] = trash_pk
    pltpu.sync_copy(vpk_st, bpk_sh.at[pos_st])

  for j in range(BPT):
    v = BPT * s + j
    pltpu.sync_copy(bpk_sh.at[pl.ds(v * CAP, CAP)], src_blk)
    pltpu.sync_copy(src_blk, pkp_hbm.at[c, v])


def _make_part():
  mesh = plsc.VectorSubcoreMesh(core_axis_name="c", subcore_axis_name="s",
                                num_cores=NC)
  return pl.kernel(
      _part_body,
      out_type=jax.ShapeDtypeStruct((NC, NV, CAP), jnp.int32),
      mesh=mesh,
      scratch_types=[
          pltpu.VMEM((CAP,), jnp.int32),
          pltpu.VMEM((CAP,), jnp.int32),
          pltpu.VMEM((128,), jnp.int32),
          pltpu.VMEM((128,), jnp.int32),
          pltpu.VMEM_SHARED((NV * CAP,), jnp.int32),
      ],
  )


def _agg_body(h_hbm, pkp_hbm, out_hbm, pk_blk,
              srcc0, srcc1, srcc2, srcc3, dstc0, dstc1, dstc2, dstc3,
              rows0, rows1, rows2, rows3,
              gsem0, gsem1, gsem2, gsem3, ssem0, ssem1, ssem2, ssem3,
              agg_sh):
  c = lax.axis_index("c")
  s = lax.axis_index("s")
  half_base = c * HALF
  srcc = (srcc0, srcc1, srcc2, srcc3)
  dstc = (dstc0, dstc1, dstc2, dstc3)
  rows = (rows0, rows1, rows2, rows3)
  gsem = (gsem0, gsem1, gsem2, gsem3)
  ssem = (ssem0, ssem1, ssem2, ssem3)

  zeros16 = jnp.zeros((16,), jnp.float32)

  def zrow(i, _):
    for j in range(8):
      rows0[i, pl.ds(j * 16, 16)] = zeros16
    return 0

  lax.fori_loop(0, K, zrow, 0)
  for q in range(4):
    pltpu.sync_copy(rows0, agg_sh.at[pl.ds(s * TBL_PER_TILE + q * K, K)])
  pltpu.sync_copy(rows0.at[pl.ds(0, TBL_PER_TILE - 4 * K)],
                  agg_sh.at[pl.ds(s * TBL_PER_TILE + 4 * K,
                                  TBL_PER_TILE - 4 * K)])
  plsc.subcore_barrier()

  def stage(g, b):
    for j in range(K // 16):
      e = pk_blk[pl.ds(g * K + j * 16, 16)]
      srcc[b][pl.ds(j * 16, 16)] = e & 0xFFFF
      dstc[b][pl.ds(j * 16, 16)] = e >> 16

  def start_gather(g, b):
    pltpu.async_copy(h_hbm.at[srcc[b]], rows[b], gsem[b])

  def wait_gather(b):
    pltpu.make_async_copy(h_hbm.at[srcc[b]], rows[b], gsem[b]).wait()

  def start_scatter(b):
    pltpu.async_copy(rows[b], agg_sh.at[dstc[b]], ssem[b], add=True)

  def wait_scatter(b):
    pltpu.make_async_copy(rows[b], agg_sh.at[dstc[b]], ssem[b]).wait()

  for j in range(BPT):
    v = BPT * s + j
    pltpu.sync_copy(pkp_hbm.at[c, v], pk_blk)
    n_ch = pk_blk[pl.ds(CNTPOS, 16)][0]

    for b in range(3):

      @pl.when(b < n_ch)
      def _():
        stage(b, b)
        start_gather(b, b)

    def quad(qq, _):
      for b in range(4):
        g = 4 * qq + b

        @pl.when(g < n_ch)
        def _():
          wait_gather(b)
          start_scatter(b)
          bb = (b + 3) % 4

          @pl.when(g >= 1)
          def _():
            wait_scatter(bb)

          @pl.when(g + 3 < n_ch)
          def _():
            stage(g + 3, bb)
            start_gather(g + 3, bb)
      return 0

    lax.fori_loop(0, (n_ch + 3) // 4, quad, 0)

    for b in range(4):

      @pl.when((n_ch >= 1) & ((n_ch - 1) % 4 == b))
      def _():
        wait_scatter(b)

  plsc.subcore_barrier()

  for q in range(4):
    pltpu.sync_copy(agg_sh.at[pl.ds(s * OUT_PER_TILE + q * K, K)], rows[q])
    pltpu.sync_copy(
        rows[q], out_hbm.at[pl.ds(half_base + s * OUT_PER_TILE + q * K, K)])


def _make_agg():
  mesh = plsc.VectorSubcoreMesh(core_axis_name="c", subcore_axis_name="s",
                                num_cores=NC)
  return pl.kernel(
      _agg_body,
      out_type=jax.ShapeDtypeStruct((N_PAD, D), jnp.float32),
      mesh=mesh,
      scratch_types=[
          pltpu.VMEM((CAP,), jnp.int32),
          pltpu.VMEM((K,), jnp.int32),
          pltpu.VMEM((K,), jnp.int32),
          pltpu.VMEM((K,), jnp.int32),
          pltpu.VMEM((K,), jnp.int32),
          pltpu.VMEM((K,), jnp.int32),
          pltpu.VMEM((K,), jnp.int32),
          pltpu.VMEM((K,), jnp.int32),
          pltpu.VMEM((K,), jnp.int32),
          pltpu.VMEM((K, D), jnp.float32),
          pltpu.VMEM((K, D), jnp.float32),
          pltpu.VMEM((K, D), jnp.float32),
          pltpu.VMEM((K, D), jnp.float32),
          pltpu.SemaphoreType.DMA,
          pltpu.SemaphoreType.DMA,
          pltpu.SemaphoreType.DMA,
          pltpu.SemaphoreType.DMA,
          pltpu.SemaphoreType.DMA,
          pltpu.SemaphoreType.DMA,
          pltpu.SemaphoreType.DMA,
          pltpu.SemaphoreType.DMA,
          pltpu.VMEM_SHARED((TBL, D), jnp.float32),
      ],
  )


def _dense_mid_body(a_ref, h_ref, wrelT_ref, wrootT_ref, b_ref, o_ref):
  y = jnp.dot(a_ref[...], wrelT_ref[...], preferred_element_type=jnp.float32)
  y += jnp.dot(h_ref[...], wrootT_ref[...], preferred_element_type=jnp.float32)
  y += b_ref[...]
  o_ref[...] = jnp.maximum(y, 0.0)


def _final_body(h_ref, wgT_ref, bg_ref, o_ref):
  o_ref[...] = (
      jnp.dot(h_ref[...], wgT_ref[...], preferred_element_type=jnp.float32)
      + bg_ref[...])


_R = 2000


def _dense_mid(agg, h, wrelT, wrootT, b2d):
  grid = (N_NODES // _R,)
  return pl.pallas_call(
      _dense_mid_body,
      grid=grid,
      in_specs=[
          pl.BlockSpec((_R, D), lambda i: (i, 0)),
          pl.BlockSpec((_R, D), lambda i: (i, 0)),
          pl.BlockSpec((D, D), lambda i: (0, 0)),
          pl.BlockSpec((D, D), lambda i: (0, 0)),
          pl.BlockSpec((1, D), lambda i: (0, 0)),
      ],
      out_specs=pl.BlockSpec((_R, D), lambda i: (i, 0)),
      out_shape=jax.ShapeDtypeStruct((N_NODES, D), jnp.float32),
  )(agg, h, wrelT, wrootT, b2d)


def _final(h, wgT, bg2d):
  grid = (N_NODES // _R,)
  return pl.pallas_call(
      _final_body,
      grid=grid,
      in_specs=[
          pl.BlockSpec((_R, D), lambda i: (i, 0)),
          pl.BlockSpec((D, D), lambda i: (0, 0)),
          pl.BlockSpec((1, D), lambda i: (0, 0)),
      ],
      out_specs=pl.BlockSpec((_R, D), lambda i: (i, 0)),
      out_shape=jax.ShapeDtypeStruct((N_NODES, D), jnp.float32),
  )(h, wgT, bg2d)


def kernel(x, edge_index, Wrel0, brel0, Wroot0, Wrel1, brel1, Wroot1, Wrel2,
           brel2, Wroot2, Wg, bg):
  n_edges = edge_index.shape[1]
  src = edge_index[0]
  dst = edge_index[1]
  pkp = _make_part()(src, dst)
  agg_fn = _make_agg()

  wrelT = jnp.stack([Wrel0.T, Wrel1.T, Wrel2.T])
  wrootT = jnp.stack([Wroot0.T, Wroot1.T, Wroot2.T])
  b2 = jnp.stack([brel0.reshape(1, D), brel1.reshape(1, D),
                  brel2.reshape(1, D)])

  def layer(h, ws):
    wrelT_i, wrootT_i, b_i = ws
    agg = agg_fn(h, pkp)
    h2 = _dense_mid(agg, h, wrelT_i, wrootT_i, b_i)
    return h2, None

  h3, _ = lax.scan(layer, x, (wrelT, wrootT, b2))
  return _final(h3, Wg.T, bg.reshape(1, D))
